# baseline (speedup 1.0000x reference)
import functools

import jax
import jax.numpy as jnp
from jax import lax
from jax.experimental import pallas as pl
from jax.experimental.pallas import tpu as pltpu
from jax.experimental.pallas import tpu_sc as plsc

B = 16
C = 512
E = 512
HW = 64 * 64
TN = 4096
NT = HW // TN
LOSS_SCALE = 1.25 / (B * HW * C)

NC = 2
L = 16


def _vq_tc_body(x_ref, w_ref, q_ref, idx_ref, loss_ref, wsq_ref):
    b = pl.program_id(0)
    t = pl.program_id(1)
    x = x_ref[0]
    w = w_ref[...]

    @pl.when(jnp.logical_and(b == 0, t == 0))
    def _wsq():
        wsq_ref[...] = jnp.sum(w * w, axis=1, keepdims=True)

    wsq = wsq_ref[...]
    xsq = jnp.sum(x * x, axis=0, keepdims=True)
    scores = (wsq + xsq) - 2.0 * jax.lax.dot_general(
        w, x, (((1,), (0,)), ((), ())), preferred_element_type=jnp.float32)
    m = jnp.min(scores, axis=0, keepdims=True)
    iota_e = jax.lax.broadcasted_iota(jnp.int32, (E, TN), 0)
    idx = jnp.min(jnp.where(scores == m, iota_e, E), axis=0, keepdims=True)
    idx_ref[0] = idx
    onehot = (iota_e == idx).astype(jnp.float32)
    q_ref[0] = jax.lax.dot_general(
        w, onehot, (((0,), (0,)), ((), ())), preferred_element_type=jnp.float32)

    @pl.when(jnp.logical_and(b == 0, t == 0))
    def _init():
        loss_ref[...] = jnp.zeros((1, 1), jnp.float32)

    loss_ref[...] += jnp.sum(m, axis=1, keepdims=True)

    @pl.when(jnp.logical_and(b == B - 1, t == NT - 1))
    def _finish():
        loss_ref[...] *= LOSS_SCALE


def _sc_perp_body(idx_hbm, cnt_hbm, idx_v, pres_v, out_v):
    wid = lax.axis_index("s") * NC + lax.axis_index("c")

    @pl.when(wid < B)
    def _work():
        b = wid
        pltpu.sync_copy(idx_hbm.at[b], idx_v)
        zeros = jnp.zeros((L,), jnp.float32)
        ones = jnp.ones((L,), jnp.float32)

        def zero_body(j, carry):
            pres_v[pl.ds(j * L, L)] = zeros
            return carry

        lax.fori_loop(0, E // L, zero_body, 0, unroll=False)

        def scat_body(i, carry):
            idx16 = idx_v[pl.ds(i * L, L)]
            plsc.store_scatter(pres_v, [idx16], ones)
            return carry

        lax.fori_loop(0, HW // L, scat_body, 0, unroll=False)

        def sum_body(j, acc):
            return acc + pres_v[pl.ds(j * L, L)]

        acc = lax.fori_loop(0, E // L, sum_body, jnp.zeros((L,), jnp.float32),
                            unroll=False)
        cnt = jnp.sum(acc)
        out_v[...] = jnp.full((L,), cnt, jnp.float32)
        pltpu.sync_copy(out_v, cnt_hbm.at[b])


@functools.partial(
    pl.kernel,
    mesh=plsc.VectorSubcoreMesh(core_axis_name="c", subcore_axis_name="s"),
    out_type=jax.ShapeDtypeStruct((B, L), jnp.float32),
    scratch_types=[
        pltpu.VMEM((HW,), jnp.int32),
        pltpu.VMEM((E,), jnp.float32),
        pltpu.VMEM((L,), jnp.float32),
    ],
    compiler_params=pltpu.CompilerParams(needs_layout_passes=False),
)
def _sc_perp(idx_hbm, cnt_hbm, idx_v, pres_v, out_v):
    _sc_perp_body(idx_hbm, cnt_hbm, idx_v, pres_v, out_v)


def kernel(inputs, weight):
    x3 = inputs.reshape(B, C, HW)
    q, idxr, loss = pl.pallas_call(
        _vq_tc_body,
        grid=(B, NT),
        in_specs=[
            pl.BlockSpec((1, C, TN), lambda b, t: (b, 0, t)),
            pl.BlockSpec((E, C), lambda b, t: (0, 0)),
        ],
        out_specs=[
            pl.BlockSpec((1, C, TN), lambda b, t: (b, 0, t)),
            pl.BlockSpec((1, 1, TN), lambda b, t: (b * NT + t, 0, 0)),
            pl.BlockSpec((1, 1), lambda b, t: (0, 0)),
        ],
        out_shape=[
            jax.ShapeDtypeStruct((B, C, HW), jnp.float32),
            jax.ShapeDtypeStruct((B * NT, 1, TN), jnp.int32),
            jax.ShapeDtypeStruct((1, 1), jnp.float32),
        ],
        scratch_shapes=[pltpu.VMEM((E, 1), jnp.float32)],
        compiler_params=pltpu.CompilerParams(
            dimension_semantics=("arbitrary", "arbitrary")),
    )(x3, weight)
    quantized_out = q.reshape(B, C, 64, 64)
    encoding_indices = idxr.reshape(B, HW)
    counts = _sc_perp(encoding_indices)
    perplexity = jnp.mean(counts[:, 0])
    return (loss[0, 0], quantized_out, perplexity, encoding_indices)


# speedup vs baseline: 1.0254x; 1.0254x over previous
import functools

import jax
import jax.numpy as jnp
from jax import lax
from jax.experimental import pallas as pl
from jax.experimental.pallas import tpu as pltpu
from jax.experimental.pallas import tpu_sc as plsc

B = 16
C = 512
E = 512
HW = 64 * 64
TN = 4096
NT = HW // TN
LOSS_SCALE = 1.25 / (B * HW * C)

NC = 2
L = 16


def _vq_tc_body(x_ref, w_ref, q_ref, idx_ref, loss_ref):
    b = pl.program_id(0)
    t = pl.program_id(1)
    x = x_ref[0]
    w = w_ref[...]
    wsq = jnp.sum(w * w, axis=1, keepdims=True)
    xsq = jnp.sum(x * x, axis=0, keepdims=True)
    scores = (wsq + xsq) - 2.0 * jax.lax.dot_general(
        w, x, (((1,), (0,)), ((), ())), preferred_element_type=jnp.float32)
    m = jnp.min(scores, axis=0, keepdims=True)
    iota_e = jax.lax.broadcasted_iota(jnp.int32, (E, TN), 0)
    idx = jnp.min(jnp.where(scores == m, iota_e, E), axis=0, keepdims=True)
    idx_ref[0] = idx
    onehot = (iota_e == idx).astype(jnp.float32)
    q_ref[0] = jax.lax.dot_general(
        w, onehot, (((0,), (0,)), ((), ())), preferred_element_type=jnp.float32)

    @pl.when(jnp.logical_and(b == 0, t == 0))
    def _init():
        loss_ref[...] = jnp.zeros((1, 1), jnp.float32)

    loss_ref[...] += jnp.sum(m, axis=1, keepdims=True)

    @pl.when(jnp.logical_and(b == B - 1, t == NT - 1))
    def _finish():
        loss_ref[...] *= LOSS_SCALE


def _sc_perp_body(idx_hbm, cnt_hbm, idx_v, pres_v, out_v):
    wid = lax.axis_index("s") * NC + lax.axis_index("c")

    @pl.when(wid < B)
    def _work():
        b = wid
        pltpu.sync_copy(idx_hbm.at[b], idx_v)
        zeros = jnp.zeros((L,), jnp.float32)
        ones = jnp.ones((L,), jnp.float32)

        def zero_body(j, carry):
            pres_v[pl.ds(j * L, L)] = zeros
            return carry

        lax.fori_loop(0, E // L, zero_body, 0, unroll=8)

        def scat_body(i, carry):
            idx16 = idx_v[pl.ds(i * L, L)]
            plsc.store_scatter(pres_v, [idx16], ones)
            return carry

        lax.fori_loop(0, HW // L, scat_body, 0, unroll=8)

        def sum_body(j, acc):
            return acc + pres_v[pl.ds(j * L, L)]

        acc = lax.fori_loop(0, E // L, sum_body, jnp.zeros((L,), jnp.float32),
                            unroll=8)
        cnt = jnp.sum(acc)
        out_v[...] = jnp.full((L,), cnt, jnp.float32)
        pltpu.sync_copy(out_v, cnt_hbm.at[b])


@functools.partial(
    pl.kernel,
    mesh=plsc.VectorSubcoreMesh(core_axis_name="c", subcore_axis_name="s"),
    out_type=jax.ShapeDtypeStruct((B, L), jnp.float32),
    scratch_types=[
        pltpu.VMEM((HW,), jnp.int32),
        pltpu.VMEM((E,), jnp.float32),
        pltpu.VMEM((L,), jnp.float32),
    ],
    compiler_params=pltpu.CompilerParams(needs_layout_passes=False),
)
def _sc_perp(idx_hbm, cnt_hbm, idx_v, pres_v, out_v):
    _sc_perp_body(idx_hbm, cnt_hbm, idx_v, pres_v, out_v)


def kernel(inputs, weight):
    x3 = inputs.reshape(B, C, HW)
    q, idxr, loss = pl.pallas_call(
        _vq_tc_body,
        grid=(B, NT),
        in_specs=[
            pl.BlockSpec((1, C, TN), lambda b, t: (b, 0, t)),
            pl.BlockSpec((E, C), lambda b, t: (0, 0)),
        ],
        out_specs=[
            pl.BlockSpec((1, C, TN), lambda b, t: (b, 0, t)),
            pl.BlockSpec((1, 1, TN), lambda b, t: (b * NT + t, 0, 0)),
            pl.BlockSpec((1, 1), lambda b, t: (0, 0)),
        ],
        out_shape=[
            jax.ShapeDtypeStruct((B, C, HW), jnp.float32),
            jax.ShapeDtypeStruct((B * NT, 1, TN), jnp.int32),
            jax.ShapeDtypeStruct((1, 1), jnp.float32),
        ],
        compiler_params=pltpu.CompilerParams(
            dimension_semantics=("arbitrary", "arbitrary")),
    )(x3, weight)
    quantized_out = q.reshape(B, C, 64, 64)
    encoding_indices = idxr.reshape(B, HW)
    counts = _sc_perp(encoding_indices)
    perplexity = jnp.mean(counts[:, 0])
    return (loss[0, 0], quantized_out, perplexity, encoding_indices)
